# baseline (speedup 1.0000x reference)
import functools

import jax
import jax.numpy as jnp
from jax import lax
from jax.experimental import pallas as pl
from jax.experimental.pallas import tpu as pltpu
from jax.experimental.pallas import tpu_sc as plsc

NUM_ENTITIES = 50000
NUM_RELATIONS = 1000
EMBED_DIM = 64
MAX_ACTIONS = 256
BATCH = 1024

NUM_WORKERS = 32
BPW = BATCH // NUM_WORKERS
QTR = 64
UNITS = BPW * 4
PAD_D = 128
GRP = 8
NGRP = BPW // GRP
UPG = 4 * GRP
AHEAD = 3


def _body(ent_hbm, relf_hbm, mask_hbm, head_hbm, rsp_hbm, esp_hbm, out_hbm,
          rel_v, head_v, rsp_v, esp_v, msk_v, eemb_v, out_v,
          gsem, hsem, wsem0, wsem1):
    cid = lax.axis_index("c")
    sid = lax.axis_index("s")
    wid = sid * 2 + cid
    base = wid * BPW
    wsems = (wsem0, wsem1)

    pltpu.sync_copy(relf_hbm, rel_v)
    pltpu.sync_copy(head_hbm.at[pl.ds(base, BPW)], head_v)

    def issue_rows(k):
        kb = lax.rem(k, 2)
        idx = head_v.at[pl.ds(pl.multiple_of(k * GRP, GRP), GRP)]
        pltpu.async_copy(rsp_hbm.at[idx], rsp_v.at[kb], hsem)
        pltpu.async_copy(esp_hbm.at[idx], esp_v.at[kb], hsem)
        pltpu.async_copy(mask_hbm.at[idx], msk_v.at[kb], hsem)

    def wait_rows():
        for _ in range(3):
            pltpu.make_async_copy(rsp_hbm.at[head_v.at[pl.ds(0, GRP)]],
                                  rsp_v.at[0], hsem).wait()

    def issue_ent(u):
        i = u // 4
        q = lax.rem(u, 4)
        kb = lax.rem(i // GRP, 2)
        j = lax.rem(i, GRP)
        pltpu.async_copy(
            ent_hbm.at[esp_v.at[kb, j, pl.ds(q * QTR, QTR)]],
            eemb_v.at[lax.rem(u, 4)], gsem)

    def wait_ent(eb):
        pltpu.make_async_copy(ent_hbm.at[esp_v.at[0, 0, pl.ds(0, QTR)]],
                              eemb_v.at[eb], gsem).wait()

    def compute_qtr(kb, j, q, eb, ob):
        def grp_body(g, _):
            a0 = pl.multiple_of(q * QTR + g * 16, 16)
            mvec = msk_v[kb, j, pl.ds(a0, 16)]
            rvec = rsp_v[kb, j, pl.ds(a0, 16)] * EMBED_DIM
            for l in range(16):
                row = g * 16 + l
                mv = jnp.full((16,), mvec[l], dtype=jnp.float32)
                r64 = rvec[l]
                for c in range(4):
                    rr = rel_v[pl.ds(r64 + c * 16, 16)]
                    ee = eemb_v[eb, row, pl.ds(c * 16, 16)]
                    out_v[ob, row, pl.ds(c * 16, 16)] = rr * mv
                    out_v[ob, row, pl.ds(EMBED_DIM + c * 16, 16)] = ee * mv
            return _

        lax.fori_loop(0, QTR // 16, grp_body, None)

    def start_write(i, q, ob):
        pltpu.async_copy(
            out_v.at[ob],
            out_hbm.at[base + i, pl.ds(q * QTR, QTR)],
            wsems[ob])

    def wait_write(ob):
        pltpu.make_async_copy(out_v.at[ob], out_hbm.at[0, pl.ds(0, QTR)],
                              wsems[ob]).wait()

    issue_rows(0)
    wait_rows()
    issue_rows(1)
    for u in range(AHEAD):
        issue_ent(u)

    def step(t, carry):
        for ob in range(2):
            u = 2 * t + ob
            i = u // 4
            q = lax.rem(u, 4)
            kb = lax.rem(i // GRP, 2)
            j = lax.rem(i, GRP)

            nu = u + AHEAD

            @pl.when(nu < UNITS)
            def _():
                r = lax.rem(nu, UPG)

                @pl.when(jnp.logical_and(r == 0, nu >= UPG))
                def _():
                    wait_rows()

                @pl.when(jnp.logical_and(r == UPG // 2,
                                         nu // UPG + 1 < NGRP))
                def _():
                    issue_rows(nu // UPG + 1)

                issue_ent(nu)

            wait_ent(lax.rem(u, 4))

            @pl.when(u >= 2)
            def _():
                wait_write(ob)

            compute_qtr(kb, j, q, lax.rem(u, 4), ob)
            start_write(i, q, ob)
        return carry

    lax.fori_loop(0, UNITS // 2, step, None)
    wait_write(0)
    wait_write(1)


@jax.jit
def _sc_call(ent_pad, rel_flat, action_mask, head, r_space, e_space):
    mesh = plsc.VectorSubcoreMesh(core_axis_name="c", subcore_axis_name="s")
    run = pl.kernel(
        _body,
        out_type=jax.ShapeDtypeStruct((BATCH, MAX_ACTIONS, 2 * EMBED_DIM),
                                      jnp.float32),
        mesh=mesh,
        scratch_types=[
            pltpu.VMEM((NUM_RELATIONS * EMBED_DIM,), jnp.float32),
            pltpu.VMEM((BPW,), jnp.int32),
            pltpu.VMEM((2, GRP, MAX_ACTIONS), jnp.int32),
            pltpu.VMEM((2, GRP, MAX_ACTIONS), jnp.int32),
            pltpu.VMEM((2, GRP, MAX_ACTIONS), jnp.float32),
            pltpu.VMEM((4, QTR, PAD_D), jnp.float32),
            pltpu.VMEM((2, QTR, 2 * EMBED_DIM), jnp.float32),
            pltpu.SemaphoreType.DMA,
            pltpu.SemaphoreType.DMA,
            pltpu.SemaphoreType.DMA,
            pltpu.SemaphoreType.DMA,
        ],
    )
    return run(ent_pad, rel_flat, action_mask, head, r_space, e_space)


def kernel(entity_table, relation_table, action_mask, head, r_space, e_space):
    head = head.astype(jnp.int32)
    ent_pad = jnp.concatenate([entity_table, entity_table], axis=1)
    rel_flat = relation_table.reshape(-1)
    return _sc_call(ent_pad, rel_flat, action_mask, head, r_space, e_space)


# speedup vs baseline: 1.0329x; 1.0329x over previous
import functools

import jax
import jax.numpy as jnp
from jax import lax
from jax.experimental import pallas as pl
from jax.experimental.pallas import tpu as pltpu
from jax.experimental.pallas import tpu_sc as plsc

NUM_ENTITIES = 50000
NUM_RELATIONS = 1000
EMBED_DIM = 64
MAX_ACTIONS = 256
BATCH = 1024

NUM_WORKERS = 32
BPW = BATCH // NUM_WORKERS
QTR = 64
UNITS = BPW * 4
PAD_D = 128
GRP = 8
NGRP = BPW // GRP
UPG = 4 * GRP
AHEAD = 3


def _body(ent_hbm, relf_hbm, mask_hbm, head_hbm, rsp_hbm, esp_hbm, out_hbm,
          rel_v, head_v, rsp_v, esp_v, msk_v, eemb_v, out_v,
          gsem, hsem, wsem0, wsem1):
    cid = lax.axis_index("c")
    sid = lax.axis_index("s")
    wid = sid * 2 + cid
    base = wid * BPW
    wsems = (wsem0, wsem1)

    pltpu.sync_copy(relf_hbm, rel_v)
    pltpu.sync_copy(head_hbm.at[pl.ds(base, BPW)], head_v)

    def issue_rows(k):
        kb = lax.rem(k, 2)
        idx = head_v.at[pl.ds(pl.multiple_of(k * GRP, GRP), GRP)]
        pltpu.async_copy(rsp_hbm.at[idx], rsp_v.at[kb], hsem)
        pltpu.async_copy(esp_hbm.at[idx], esp_v.at[kb], hsem)
        pltpu.async_copy(mask_hbm.at[idx], msk_v.at[kb], hsem)

    def wait_rows():
        for _ in range(3):
            pltpu.make_async_copy(rsp_hbm.at[head_v.at[pl.ds(0, GRP)]],
                                  rsp_v.at[0], hsem).wait()

    def issue_ent(u):
        i = u // 4
        q = lax.rem(u, 4)
        kb = lax.rem(i // GRP, 2)
        j = lax.rem(i, GRP)
        pltpu.async_copy(
            ent_hbm.at[esp_v.at[kb, j, pl.ds(q * QTR, QTR)]],
            eemb_v.at[lax.rem(u, 4)], gsem)

    def wait_ent(eb):
        pltpu.make_async_copy(ent_hbm.at[esp_v.at[0, 0, pl.ds(0, QTR)]],
                              eemb_v.at[eb], gsem).wait()

    def compute_qtr(kb, j, q, eb, ob):
        def grp_body(g, _):
            a0 = pl.multiple_of(q * QTR + g * 16, 16)
            mvec = msk_v[kb, j, pl.ds(a0, 16)]
            rvec = rsp_v[kb, j, pl.ds(a0, 16)] * EMBED_DIM
            for l in range(16):
                row = g * 16 + l
                mv = jnp.full((16,), mvec[l], dtype=jnp.float32)
                r64 = rvec[l]
                for c in range(4):
                    rr = rel_v[pl.ds(r64 + c * 16, 16)]
                    ee = eemb_v[eb, row, pl.ds(c * 16, 16)]
                    out_v[ob, row, pl.ds(c * 16, 16)] = rr * mv
                    out_v[ob, row, pl.ds(EMBED_DIM + c * 16, 16)] = ee * mv
            return _

        lax.fori_loop(0, QTR // 16, grp_body, None)

    def start_write(i, q, ob):
        pltpu.async_copy(
            out_v.at[ob],
            out_hbm.at[base + i, pl.ds(q * QTR, QTR)],
            wsems[ob])

    def wait_write(ob):
        pltpu.make_async_copy(out_v.at[ob], out_hbm.at[0, pl.ds(0, QTR)],
                              wsems[ob]).wait()

    issue_rows(0)
    wait_rows()
    issue_rows(1)
    for u in range(AHEAD):
        issue_ent(u)

    def step(t, carry):
        for ob in range(2):
            u = 2 * t + ob
            i = u // 4
            q = lax.rem(u, 4)
            kb = lax.rem(i // GRP, 2)
            j = lax.rem(i, GRP)

            nu = u + AHEAD

            @pl.when(nu < UNITS)
            def _():
                r = lax.rem(nu, UPG)

                @pl.when(jnp.logical_and(r == 0, nu >= UPG))
                def _():
                    wait_rows()

                @pl.when(jnp.logical_and(r == UPG // 2,
                                         nu // UPG + 1 < NGRP))
                def _():
                    issue_rows(nu // UPG + 1)

                issue_ent(nu)

            wait_ent(lax.rem(u, 4))

            @pl.when(u >= 2)
            def _():
                wait_write(ob)

            compute_qtr(kb, j, q, lax.rem(u, 4), ob)
            start_write(i, q, ob)
        return carry

    lax.fori_loop(0, UNITS // 2, step, None)
    wait_write(0)
    wait_write(1)


@jax.jit
def _sc_call(ent_pad, rel_flat, action_mask, head, r_space, e_space):
    mesh = plsc.VectorSubcoreMesh(core_axis_name="c", subcore_axis_name="s")
    run = pl.kernel(
        _body,
        out_type=jax.ShapeDtypeStruct((BATCH, MAX_ACTIONS, 2 * EMBED_DIM),
                                      jnp.float32),
        mesh=mesh,
        scratch_types=[
            pltpu.VMEM((NUM_RELATIONS * EMBED_DIM,), jnp.float32),
            pltpu.VMEM((BPW,), jnp.int32),
            pltpu.VMEM((2, GRP, MAX_ACTIONS), jnp.int32),
            pltpu.VMEM((2, GRP, MAX_ACTIONS), jnp.int32),
            pltpu.VMEM((2, GRP, MAX_ACTIONS), jnp.float32),
            pltpu.VMEM((4, QTR, PAD_D), jnp.float32),
            pltpu.VMEM((2, QTR, 2 * EMBED_DIM), jnp.float32),
            pltpu.SemaphoreType.DMA,
            pltpu.SemaphoreType.DMA,
            pltpu.SemaphoreType.DMA,
            pltpu.SemaphoreType.DMA,
        ],
    )
    return run(ent_pad, rel_flat, action_mask, head, r_space, e_space)


def kernel(entity_table, relation_table, action_mask, head, r_space, e_space):
    head = head.astype(jnp.int32)
    ent_pad = jnp.pad(entity_table, ((0, 0), (0, PAD_D - EMBED_DIM)))
    rel_flat = relation_table.reshape(-1)
    return _sc_call(ent_pad, rel_flat, action_mask, head, r_space, e_space)
